# baseline (speedup 1.0000x reference)
import functools
import jax
import jax.numpy as jnp
from jax import lax
from jax.experimental import pallas as pl
from jax.experimental.pallas import tpu as pltpu
from jax.experimental.pallas import tpu_sc as plsc

N = 10000
E2 = 160000
E = 2 * E2
ML = 64
DIA = 2
ZETA = 1e-6

EBLK = 8000
NBLK = 10000

NC, NS = 2, 16
NW = NC * NS
GCHUNK = 400
HNP = 10240
ECH = 2000
NEG_INF = float("-inf")


def _sc_gather_body(table_hbm, idx_hbm, out_hbm, idx_v, rows_v, sem):
    wid = lax.axis_index("s") * NC + lax.axis_index("c")
    per_w = E // NW
    base = wid * per_w
    nch = per_w // GCHUNK
    pltpu.sync_copy(idx_hbm.at[pl.ds(base, GCHUNK)], idx_v.at[0])
    descs = [pltpu.async_copy(table_hbm.at[idx_v.at[0]], rows_v.at[0], sem)]
    for c in range(nch):
        b = c & 1
        if c + 1 < nch:
            nb = 1 - b
            off = base + (c + 1) * GCHUNK
            pltpu.sync_copy(idx_hbm.at[pl.ds(off, GCHUNK)], idx_v.at[nb])
            descs.append(pltpu.async_copy(table_hbm.at[idx_v.at[nb]],
                                          rows_v.at[nb], sem))
        descs[c].wait()
        pltpu.sync_copy(rows_v.at[b], out_hbm.at[pl.ds(base + c * GCHUNK,
                                                       GCHUNK)])


@jax.jit
def _sc_gather(table, idx):
    mesh = plsc.VectorSubcoreMesh(core_axis_name="c", subcore_axis_name="s")
    return pl.kernel(
        _sc_gather_body,
        out_type=jax.ShapeDtypeStruct((E, ML), jnp.float32),
        mesh=mesh,
        scratch_types=[
            pltpu.VMEM((2, GCHUNK), jnp.int32),
            pltpu.VMEM((2, GCHUNK, ML), jnp.float32),
            pltpu.SemaphoreType.DMA,
        ],
        compiler_params=pltpu.CompilerParams(use_tc_tiling_on_sc=False, needs_layout_passes=False),
    )(table, idx)


SCAP = 16384
SNR = HNP // NW
MCH = 992


def _segmax_body(m_hbm, dst_hbm, out_hbm, acc_v, dst_v, elist_v, dloc_v,
                 rows_v, sem):
    cid = lax.axis_index("c")
    sid = lax.axis_index("s")
    w = sid * NC + cid
    lo = w * SNR
    hi = lo + SNR

    def init_acc(i, _):
        for jj in range(4):
            acc_v[i, pl.ds(jj * 16, 16)] = jnp.full((16,), NEG_INF,
                                                    jnp.float32)
        return _

    lax.fori_loop(0, SNR + 8, init_acc, 0)

    def init_list(i, _):
        sl = pl.ds(i * 16, 16)
        elist_v[sl] = jnp.zeros((16,), jnp.int32)
        dloc_v[sl] = jnp.full((16,), SNR, jnp.int32)
        return _

    lax.fori_loop(0, SCAP // 16, init_list, 0)

    iot = lax.iota(jnp.int32, 16)

    def scan_chunk(c, cnt):
        pltpu.sync_copy(dst_hbm.at[pl.ds(c * ECH, ECH)], dst_v)

        def scan_grp(j, cnt):
            sl = pl.ds(j * 16, 16)
            dv = dst_v[sl]
            msk = (dv >= lo) & (dv < hi)
            ids = c * ECH + j * 16 + iot
            inc = msk.astype(jnp.int32)
            csum = plsc.cumsum(inc)
            pos = jnp.minimum(cnt, SCAP - 16) + csum - inc
            plsc.store_scatter(elist_v, [pos], ids, mask=msk)
            plsc.store_scatter(dloc_v, [pos], dv - lo, mask=msk)
            return cnt + jnp.sum(inc)

        return lax.fori_loop(0, ECH // 16, scan_grp, cnt)

    cnt = lax.fori_loop(0, E // ECH, scan_chunk, jnp.int32(0))
    cnt = jnp.minimum(cnt, SCAP)
    nch = (cnt + MCH - 1) // MCH

    def proc_chunk(q, _):
        pltpu.async_copy(m_hbm.at[elist_v.at[pl.ds(q * MCH, MCH)]],
                         rows_v, sem).wait()

        def rmw(t16, _):
            dlv = dloc_v[pl.ds(q * MCH + t16 * 16, 16)]
            for t in range(16):
                dl = dlv[t]
                for jj in range(4):
                    sl = pl.ds(jj * 16, 16)
                    acc_v[dl, sl] = jnp.maximum(acc_v[dl, sl],
                                                rows_v[t16 * 16 + t, sl])
            return _

        lax.fori_loop(0, MCH // 16, rmw, 0)
        return _

    lax.fori_loop(0, nch, proc_chunk, 0)

    def clean(i, _):
        for jj in range(4):
            sl = pl.ds(jj * 16, 16)
            v = acc_v[i, sl]
            rows_v[i, sl] = jnp.where(v == NEG_INF, 0.0, v)
        return _

    lax.fori_loop(0, SNR, clean, 0)
    pltpu.sync_copy(rows_v.at[pl.ds(0, SNR)], out_hbm.at[pl.ds(lo, SNR)])


@jax.jit
def _sc_segmax(m, dst):
    mesh = plsc.VectorSubcoreMesh(core_axis_name="c", subcore_axis_name="s")
    return pl.kernel(
        _segmax_body,
        out_type=jax.ShapeDtypeStruct((HNP, ML), jnp.float32),
        mesh=mesh,
        scratch_types=[
            pltpu.VMEM((SNR + 8, ML), jnp.float32),
            pltpu.VMEM((ECH,), jnp.int32),
            pltpu.VMEM((SCAP,), jnp.int32),
            pltpu.VMEM((SCAP,), jnp.int32),
            pltpu.VMEM((MCH, ML), jnp.float32),
            pltpu.SemaphoreType.DMA,
        ],
        compiler_params=pltpu.CompilerParams(use_tc_tiling_on_sc=False, needs_layout_passes=False),
    )(m, dst)


_TAKE_DN = lax.GatherDimensionNumbers(
    offset_dims=(), collapsed_slice_dims=(0,), start_index_map=(0,))


def _vtake(x, i):
    return lax.gather(x, i[:, None], _TAKE_DN, (1,),
                      mode=lax.GatherScatterMode.PROMISE_IN_BOUNDS)


def _seg_scan_last(key, val, neutral, op):
    iot = lax.iota(jnp.int32, 16)
    for s in (1, 2, 4, 8):
        pidx = jnp.maximum(iot - s, 0)
        pk = _vtake(key, pidx)
        pv = _vtake(val, pidx)
        ok = (pk == key) & (iot >= s)
        val = op(val, jnp.where(ok, pv, neutral))
    nk = _vtake(key, jnp.minimum(iot + 1, 15))
    last = (key != nk) | (iot == 15)
    return val, last


def _combine_partials(acc_v, shared, red_v, cmb_v, part_hbm, cid, sid, op):
    pltpu.sync_copy(acc_v, shared.at[sid])
    plsc.subcore_barrier()
    base = sid * 640
    pltpu.sync_copy(shared.at[0, pl.ds(base, 640)], red_v)
    for t in range(1, 16):
        pltpu.sync_copy(shared.at[t, pl.ds(base, 640)], cmb_v)

        def red_body(i, _):
            sl = pl.ds(i * 16, 16)
            red_v[sl] = op(red_v[sl], cmb_v[sl])
            return _

        lax.fori_loop(0, 40, red_body, 0)
    pltpu.sync_copy(red_v, part_hbm.at[cid, pl.ds(base, 640)])


def _heads_step_body(h_hbm, kn_hbm, lh_hbm, src_hbm, dst_hbm, part_hbm,
                     h_v, kn_v, acc_v, src_v, dst_v, l_v, red_v, cmb_v,
                     shared):
    cid = lax.axis_index("c")
    sid = lax.axis_index("s")
    w = sid * NC + cid
    pltpu.sync_copy(h_hbm, h_v)
    pltpu.sync_copy(kn_hbm, kn_v)

    def init_body(i, _):
        acc_v[pl.ds(i * 16, 16)] = jnp.full((16,), NEG_INF, jnp.float32)
        return _

    lax.fori_loop(0, HNP // 16, init_body, 0)

    base = w * (E // NW)
    is_h1 = base < E2
    sign = jnp.where(is_h1, 1.0, -1.0).astype(jnp.float32)
    loff = base - jnp.where(is_h1, 0, E2)
    for c in range(E // NW // ECH):
        pltpu.sync_copy(src_hbm.at[pl.ds(base + c * ECH, ECH)], src_v)
        pltpu.sync_copy(dst_hbm.at[pl.ds(base + c * ECH, ECH)], dst_v)
        pltpu.sync_copy(lh_hbm.at[pl.ds(loff + c * ECH, ECH)], l_v)

        def grp_body(j, _):
            sl = pl.ds(j * 16, 16)
            sv = src_v[sl]
            dv = dst_v[sl]
            lv = l_v[sl] * sign
            hs = plsc.load_gather(h_v, [sv])
            ks = plsc.load_gather(kn_v, [sv])
            cand = jnp.where(ks > 0, hs - lv, NEG_INF)
            key, val = plsc.sort_key_val(dv, cand)
            val, last = _seg_scan_last(key, val, NEG_INF, jnp.maximum)
            cur = plsc.load_gather(acc_v, [key])
            plsc.store_scatter(acc_v, [key], jnp.maximum(cur, val), mask=last)
            return _

        lax.fori_loop(0, ECH // 16, grp_body, 0)
    _combine_partials(acc_v, shared, red_v, cmb_v, part_hbm, cid, sid,
                      jnp.maximum)


@jax.jit
def _sc_heads(h, kn, lh, src, dst):
    mesh = plsc.VectorSubcoreMesh(core_axis_name="c", subcore_axis_name="s")
    return pl.kernel(
        _heads_step_body,
        out_type=jax.ShapeDtypeStruct((2, HNP), jnp.float32),
        mesh=mesh,
        scratch_types=[
            pltpu.VMEM((N,), jnp.float32),
            pltpu.VMEM((N,), jnp.float32),
            pltpu.VMEM((HNP,), jnp.float32),
            pltpu.VMEM((ECH,), jnp.int32),
            pltpu.VMEM((ECH,), jnp.int32),
            pltpu.VMEM((ECH,), jnp.float32),
            pltpu.VMEM((640,), jnp.float32),
            pltpu.VMEM((640,), jnp.float32),
            pltpu.VMEM_SHARED((16, HNP), jnp.float32),
        ],
        compiler_params=pltpu.CompilerParams(use_tc_tiling_on_sc=False, needs_layout_passes=False),
    )(h, kn, lh, src, dst)


def _dhat_body(qd_hbm, dst_hbm, part_hbm,
               acc_v, dst_v, q_v, red_v, cmb_v, shared):
    cid = lax.axis_index("c")
    sid = lax.axis_index("s")
    w = sid * NC + cid

    def init_body(i, _):
        acc_v[pl.ds(i * 16, 16)] = jnp.zeros((16,), jnp.float32)
        return _

    lax.fori_loop(0, HNP // 16, init_body, 0)

    base = w * (E // NW)
    is_h1 = base < E2
    sign = jnp.where(is_h1, 1.0, -1.0).astype(jnp.float32)
    qoff = base - jnp.where(is_h1, 0, E2)
    for c in range(E // NW // ECH):
        pltpu.sync_copy(dst_hbm.at[pl.ds(base + c * ECH, ECH)], dst_v)
        pltpu.sync_copy(qd_hbm.at[pl.ds(qoff + c * ECH, ECH)], q_v)

        def grp_body(j, _):
            sl = pl.ds(j * 16, 16)
            dv = dst_v[sl]
            qv = q_v[sl] * sign
            key, val = plsc.sort_key_val(dv, qv)
            val, last = _seg_scan_last(key, val, 0.0, jnp.add)
            cur = plsc.load_gather(acc_v, [key])
            plsc.store_scatter(acc_v, [key], cur + val, mask=last)
            return _

        lax.fori_loop(0, ECH // 16, grp_body, 0)
    _combine_partials(acc_v, shared, red_v, cmb_v, part_hbm, cid, sid,
                      jnp.add)


@jax.jit
def _sc_dhat(qd, dst):
    mesh = plsc.VectorSubcoreMesh(core_axis_name="c", subcore_axis_name="s")
    return pl.kernel(
        _dhat_body,
        out_type=jax.ShapeDtypeStruct((2, HNP), jnp.float32),
        mesh=mesh,
        scratch_types=[
            pltpu.VMEM((HNP,), jnp.float32),
            pltpu.VMEM((ECH,), jnp.int32),
            pltpu.VMEM((ECH,), jnp.float32),
            pltpu.VMEM((640,), jnp.float32),
            pltpu.VMEM((640,), jnp.float32),
            pltpu.VMEM_SHARED((16, HNP), jnp.float32),
        ],
        compiler_params=pltpu.CompilerParams(use_tc_tiling_on_sc=False, needs_layout_passes=False),
    )(qd, dst)


def _netflow_body(h_hbm, src_hbm, dst_hbm, dh_hbm, h_v, src_v, dst_v, dh_v):
    cid = lax.axis_index("c")
    sid = lax.axis_index("s")
    w = sid * NC + cid

    @pl.when(w < 16)
    def _():
        pltpu.sync_copy(h_hbm, h_v)
        base = w * (E2 // 16)
        for c in range(E2 // 16 // ECH):
            pltpu.sync_copy(src_hbm.at[pl.ds(base + c * ECH, ECH)], src_v)
            pltpu.sync_copy(dst_hbm.at[pl.ds(base + c * ECH, ECH)], dst_v)

            def grp_body(j, _):
                sl = pl.ds(j * 16, 16)
                hs = plsc.load_gather(h_v, [src_v[sl]])
                hd = plsc.load_gather(h_v, [dst_v[sl]])
                dh_v[sl] = hs - hd
                return _

            lax.fori_loop(0, ECH // 16, grp_body, 0)
            pltpu.sync_copy(dh_v, dh_hbm.at[pl.ds(base + c * ECH, ECH)])


@jax.jit
def _sc_netflow(h, src, dst):
    mesh = plsc.VectorSubcoreMesh(core_axis_name="c", subcore_axis_name="s")
    return pl.kernel(
        _netflow_body,
        out_type=jax.ShapeDtypeStruct((E2,), jnp.float32),
        mesh=mesh,
        scratch_types=[
            pltpu.VMEM((N,), jnp.float32),
            pltpu.VMEM((ECH,), jnp.int32),
            pltpu.VMEM((ECH,), jnp.int32),
            pltpu.VMEM((ECH,), jnp.float32),
        ],
        compiler_params=pltpu.CompilerParams(use_tc_tiling_on_sc=False, needs_layout_passes=False),
    )(h, src, dst)


def _edge_mlp_body(gs_ref, gd_ref, z_ref, a_ref, b_ref, c_ref, w2_ref, m_ref):
    pre = (gs_ref[...] @ a_ref[...] + gd_ref[...] @ b_ref[...]
           + z_ref[...] @ c_ref[...])
    m_ref[...] = jnp.maximum(pre, 0.0) @ w2_ref[...]


def _edge_mlp(gs, z, ew1, ew2):
    a, b, c = ew1[:ML], ew1[ML:2 * ML], ew1[2 * ML:]
    grid = E // EBLK
    g2 = grid // 2
    return pl.pallas_call(
        _edge_mlp_body,
        grid=(grid,),
        in_specs=[
            pl.BlockSpec((EBLK, ML), lambda i: (i, 0)),
            pl.BlockSpec((EBLK, ML), lambda i: ((i + g2) % grid, 0)),
            pl.BlockSpec((EBLK, ML), lambda i: (i, 0)),
            pl.BlockSpec((ML, ML), lambda i: (0, 0)),
            pl.BlockSpec((ML, ML), lambda i: (0, 0)),
            pl.BlockSpec((ML, ML), lambda i: (0, 0)),
            pl.BlockSpec((ML, ML), lambda i: (0, 0)),
        ],
        out_specs=pl.BlockSpec((EBLK, ML), lambda i: (i, 0)),
        out_shape=jax.ShapeDtypeStruct((E, ML), jnp.float32),
    )(gs, gs, z, a, b, c, ew2)


def _edge_mlp_noz_body(gs_ref, gd_ref, a_ref, b_ref, w2_ref, m_ref):
    pre = gs_ref[...] @ a_ref[...] + gd_ref[...] @ b_ref[...]
    m_ref[...] = jnp.maximum(pre, 0.0) @ w2_ref[...]


def _edge_mlp_noz(gs, ew1, ew2):
    a, b = ew1[:ML], ew1[ML:2 * ML]
    grid = E // EBLK
    g2 = grid // 2
    return pl.pallas_call(
        _edge_mlp_noz_body,
        grid=(grid,),
        in_specs=[
            pl.BlockSpec((EBLK, ML), lambda i: (i, 0)),
            pl.BlockSpec((EBLK, ML), lambda i: ((i + g2) % grid, 0)),
            pl.BlockSpec((ML, ML), lambda i: (0, 0)),
            pl.BlockSpec((ML, ML), lambda i: (0, 0)),
            pl.BlockSpec((ML, ML), lambda i: (0, 0)),
        ],
        out_specs=pl.BlockSpec((EBLK, ML), lambda i: (i, 0)),
        out_shape=jax.ShapeDtypeStruct((E, ML), jnp.float32),
    )(gs, gs, a, b, ew2)


def _node_mlp_body(g_ref, agg_ref, n1_ref, n2_ref, w2_ref, out_ref):
    pre = g_ref[...] @ n1_ref[...] + agg_ref[...] @ n2_ref[...]
    out_ref[...] = jnp.maximum(pre, 0.0) @ w2_ref[...]


def _node_mlp(g, agg, nw1, nw2):
    n1, n2 = nw1[:ML], nw1[ML:]
    grid = N // NBLK
    return pl.pallas_call(
        _node_mlp_body,
        grid=(grid,),
        in_specs=[
            pl.BlockSpec((NBLK, ML), lambda i: (i, 0)),
            pl.BlockSpec((NBLK, ML), lambda i: (i, 0)),
            pl.BlockSpec((ML, ML), lambda i: (0, 0)),
            pl.BlockSpec((ML, ML), lambda i: (0, 0)),
            pl.BlockSpec((ML, ML), lambda i: (0, 0)),
        ],
        out_specs=pl.BlockSpec((NBLK, ML), lambda i: (i, 0)),
        out_shape=jax.ShapeDtypeStruct((N, ML), jnp.float32),
    )(g, agg, n1, n2, nw2)


def _zbar_body(gs_ref, gd_ref, z_ref, z1_ref, z2_ref, z3_ref, f1_ref, f2_ref,
               c1_ref, c2_ref):
    zbar = (gs_ref[...] @ z1_ref[...] + gd_ref[...] @ z2_ref[...]
            + z_ref[...] @ z3_ref[...])
    c1_ref[...] = zbar @ f1_ref[...]
    c2_ref[...] = zbar @ f2_ref[...]


def _zbar_flows(gs, z, z_latent_W, flows_latent_W):
    z1, z2, z3 = z_latent_W[:ML], z_latent_W[ML:2 * ML], z_latent_W[2 * ML:]
    f1, f2 = flows_latent_W[:ML], flows_latent_W[ML:]
    grid = E // EBLK
    g2 = grid // 2
    c1, c2 = pl.pallas_call(
        _zbar_body,
        grid=(grid,),
        in_specs=[
            pl.BlockSpec((EBLK, ML), lambda i: (i, 0)),
            pl.BlockSpec((EBLK, ML), lambda i: ((i + g2) % grid, 0)),
            pl.BlockSpec((EBLK, ML), lambda i: (i, 0)),
            pl.BlockSpec((ML, ML), lambda i: (0, 0)),
            pl.BlockSpec((ML, ML), lambda i: (0, 0)),
            pl.BlockSpec((ML, ML), lambda i: (0, 0)),
            pl.BlockSpec((ML, 1), lambda i: (0, 0)),
            pl.BlockSpec((ML, 1), lambda i: (0, 0)),
        ],
        out_specs=[
            pl.BlockSpec((EBLK, 1), lambda i: (i, 0)),
            pl.BlockSpec((EBLK, 1), lambda i: (i, 0)),
        ],
        out_shape=[
            jax.ShapeDtypeStruct((E, 1), jnp.float32),
            jax.ShapeDtypeStruct((E, 1), jnp.float32),
        ],
    )(gs, gs, z, z1, z2, z3, f1, f2)
    return c1, c2


def _g_encode_body(p0_ref, p1_ref, ds_ref, res_ref, w_ref, g_ref):
    dhat = p0_ref[...] + p1_ref[...]
    g_ref[...] = (dhat @ w_ref[0:1, :] + ds_ref[...] @ w_ref[1:2, :]
                  + res_ref[...] @ w_ref[2:3, :])


def _g_encode(p0, p1, d_star, res, node_in_W):
    grid = N // NBLK
    return pl.pallas_call(
        _g_encode_body,
        grid=(grid,),
        in_specs=[
            pl.BlockSpec((NBLK, 1), lambda i: (i, 0)),
            pl.BlockSpec((NBLK, 1), lambda i: (i, 0)),
            pl.BlockSpec((NBLK, 1), lambda i: (i, 0)),
            pl.BlockSpec((NBLK, 1), lambda i: (i, 0)),
            pl.BlockSpec((3, ML), lambda i: (0, 0)),
        ],
        out_specs=pl.BlockSpec((NBLK, ML), lambda i: (i, 0)),
        out_shape=jax.ShapeDtypeStruct((N, ML), jnp.float32),
    )(p0, p1, d_star, res, node_in_W)


def _qd_l_body(qd_ref, c1_ref, c2_ref, r_ref, qd_out, lh_out):
    q = qd_ref[...] + c1_ref[...] + c2_ref[...]
    qd_out[...] = q
    lh_out[...] = r_ref[...] * q * jnp.power(jnp.abs(q) + ZETA, 0.852)


def _qd_l(qd, c1, c2, r_half):
    grid = E2 // EBLK
    return pl.pallas_call(
        _qd_l_body,
        grid=(grid,),
        in_specs=[
            pl.BlockSpec((EBLK, 1), lambda i: (i, 0)),
            pl.BlockSpec((EBLK, 1), lambda i: (i, 0)),
            pl.BlockSpec((EBLK, 1), lambda i: (i + grid, 0)),
            pl.BlockSpec((EBLK, 1), lambda i: (i, 0)),
        ],
        out_specs=[
            pl.BlockSpec((EBLK, 1), lambda i: (i, 0)),
            pl.BlockSpec((EBLK, 1), lambda i: (i, 0)),
        ],
        out_shape=[
            jax.ShapeDtypeStruct((E2, 1), jnp.float32),
            jax.ShapeDtypeStruct((E2, 1), jnp.float32),
        ],
    )(qd, c1, c2, r_half)


def _edge_mlp_qz_body(gs_ref, gd_ref, dh_ref, qd_ref, r_ref, a_ref, b_ref,
                      wz_ref, w2_ref, m_ref, *, g2):
    sgn = jnp.where(pl.program_id(0) < g2, 1.0, -1.0).astype(jnp.float32)
    dh = dh_ref[...]
    qt = jnp.sign(dh) * jnp.power(jnp.abs(dh) / r_ref[...] + ZETA, 1.0 / 1.852)
    z = sgn * (qt @ wz_ref[0:1, :] + qd_ref[...] @ wz_ref[1:2, :])
    pre = gs_ref[...] @ a_ref[...] + gd_ref[...] @ b_ref[...] + z
    m_ref[...] = jnp.maximum(pre, 0.0) @ w2_ref[...]


def _edge_mlp_qz(gs, dh, qd, r_half, ew1, ew2, wz):
    a, b = ew1[:ML], ew1[ML:2 * ML]
    grid = E // EBLK
    g2 = grid // 2
    return pl.pallas_call(
        functools.partial(_edge_mlp_qz_body, g2=g2),
        grid=(grid,),
        in_specs=[
            pl.BlockSpec((EBLK, ML), lambda i: (i, 0)),
            pl.BlockSpec((EBLK, ML), lambda i: ((i + g2) % grid, 0)),
            pl.BlockSpec((EBLK, 1), lambda i: (i % g2, 0)),
            pl.BlockSpec((EBLK, 1), lambda i: (i % g2, 0)),
            pl.BlockSpec((EBLK, 1), lambda i: (i % g2, 0)),
            pl.BlockSpec((ML, ML), lambda i: (0, 0)),
            pl.BlockSpec((ML, ML), lambda i: (0, 0)),
            pl.BlockSpec((2, ML), lambda i: (0, 0)),
            pl.BlockSpec((ML, ML), lambda i: (0, 0)),
        ],
        out_specs=pl.BlockSpec((EBLK, ML), lambda i: (i, 0)),
        out_shape=jax.ShapeDtypeStruct((E, ML), jnp.float32),
    )(gs, gs, dh, qd, r_half, a, b, wz, ew2)


def _head_update_body(h_ref, kn_ref, p0_ref, p1_ref, h_out, kn_out):
    seg = jnp.maximum(p0_ref[...], p1_ref[...])
    has = seg != NEG_INF
    h_out[...] = jnp.where(kn_ref[...] > 0, h_ref[...],
                           jnp.where(has, seg, 0.0))
    kn_out[...] = jnp.maximum(kn_ref[...], has.astype(jnp.float32))


def _head_update(h, kn, part):
    p0 = part[0, :N].reshape(N, 1)
    p1 = part[1, :N].reshape(N, 1)
    grid = N // NBLK
    return pl.pallas_call(
        _head_update_body,
        grid=(grid,),
        in_specs=[pl.BlockSpec((NBLK, 1), lambda i: (i, 0))] * 4,
        out_specs=[pl.BlockSpec((NBLK, 1), lambda i: (i, 0))] * 2,
        out_shape=[
            jax.ShapeDtypeStruct((N, 1), jnp.float32),
            jax.ShapeDtypeStruct((N, 1), jnp.float32),
        ],
    )(h, kn, p0, p1)


def kernel(x, edge_index, edge_attr, node_in_W, edge_W, z_latent_W,
           flows_latent_W, gnn_params):
    src = edge_index[0]
    dst = edge_index[1]
    r_half = edge_attr[:E2, 0:1]
    h_star = x[:, 0:1]
    d_star = x[:, 1:2]
    res_mask = x[:, 4:5]
    zeros_n1 = jnp.zeros((N, 1), jnp.float32)
    wz = edge_W @ gnn_params[0][0][2 * ML:]

    qd = jnp.zeros((E2, 1), jnp.float32)
    part_d = None
    dh = None
    h = h_star
    for k in range(2):
        if k == 0:
            g = _g_encode(zeros_n1, zeros_n1, d_star, res_mask, node_in_W)
        else:
            g = _g_encode(part_d[0, :N].reshape(N, 1),
                          part_d[1, :N].reshape(N, 1),
                          d_star, res_mask, node_in_W)
        z = None
        for li, (ew1, ew2, nw1, nw2) in enumerate(gnn_params):
            gs = _sc_gather(g, src)
            if li == 0:
                if k == 0:
                    m = _edge_mlp_noz(gs, ew1, ew2)
                else:
                    m = _edge_mlp_qz(gs, dh.reshape(E2, 1), qd, r_half,
                                     ew1, ew2, wz)
            else:
                m = _edge_mlp(gs, z, ew1, ew2)
            agg = jax.ops.segment_max(m, dst, num_segments=N)
            agg = jnp.where(jnp.isfinite(agg), agg, 0.0)
            g = _node_mlp(g, agg, nw1, nw2)
            z = m
        c1, c2 = _zbar_flows(_sc_gather(g, src), z, z_latent_W, flows_latent_W)
        qd, lh = _qd_l(qd, c1, c2, r_half)
        h, kn = h_star, res_mask
        for _ in range(DIA):
            part = _sc_heads(h.reshape(N), kn.reshape(N), lh.reshape(E2),
                             src, dst)
            h, kn = _head_update(h, kn, part)
        if k == 0:
            part_d = _sc_dhat(qd.reshape(E2), dst)
            dh = _sc_netflow(h.reshape(N), src, dst)
    return h


# speedup vs baseline: 1.0502x; 1.0502x over previous
import functools
import jax
import jax.numpy as jnp
from jax import lax
from jax.experimental import pallas as pl
from jax.experimental.pallas import tpu as pltpu
from jax.experimental.pallas import tpu_sc as plsc

N = 10000
E2 = 160000
E = 2 * E2
ML = 64
DIA = 2
ZETA = 1e-6

EBLK = 8000
NBLK = 10000

NC, NS = 2, 16
NW = NC * NS
GCHUNK = 400
HNP = 10240
ECH = 2000
NEG_INF = float("-inf")


def _sc_gather_body(table_hbm, idx_hbm, out_hbm, idx_v, rows_v, sem):
    wid = lax.axis_index("s") * NC + lax.axis_index("c")
    per_w = E // NW
    base = wid * per_w
    nch = per_w // GCHUNK
    pltpu.sync_copy(idx_hbm.at[pl.ds(base, GCHUNK)], idx_v.at[0])
    descs = [pltpu.async_copy(table_hbm.at[idx_v.at[0]], rows_v.at[0], sem)]
    for c in range(nch):
        b = c & 1
        if c + 1 < nch:
            nb = 1 - b
            off = base + (c + 1) * GCHUNK
            pltpu.sync_copy(idx_hbm.at[pl.ds(off, GCHUNK)], idx_v.at[nb])
            descs.append(pltpu.async_copy(table_hbm.at[idx_v.at[nb]],
                                          rows_v.at[nb], sem))
        descs[c].wait()
        pltpu.sync_copy(rows_v.at[b], out_hbm.at[pl.ds(base + c * GCHUNK,
                                                       GCHUNK)])


@jax.jit
def _sc_gather(table, idx):
    mesh = plsc.VectorSubcoreMesh(core_axis_name="c", subcore_axis_name="s")
    return pl.kernel(
        _sc_gather_body,
        out_type=jax.ShapeDtypeStruct((E, ML), jnp.float32),
        mesh=mesh,
        scratch_types=[
            pltpu.VMEM((2, GCHUNK), jnp.int32),
            pltpu.VMEM((2, GCHUNK, ML), jnp.float32),
            pltpu.SemaphoreType.DMA,
        ],
        compiler_params=pltpu.CompilerParams(use_tc_tiling_on_sc=False, needs_layout_passes=False),
    )(table, idx)


SCAP = 16384
SNR = HNP // NW
MCH = 992


def _segmax_body(m_hbm, dst_hbm, out_hbm, acc_v, dst_v, elist_v, dloc_v,
                 rows_v, sem):
    cid = lax.axis_index("c")
    sid = lax.axis_index("s")
    w = sid * NC + cid
    lo = w * SNR
    hi = lo + SNR

    def init_acc(i, _):
        for jj in range(4):
            acc_v[i, pl.ds(jj * 16, 16)] = jnp.full((16,), NEG_INF,
                                                    jnp.float32)
        return _

    lax.fori_loop(0, SNR + 8, init_acc, 0)

    def init_list(i, _):
        sl = pl.ds(i * 16, 16)
        elist_v[sl] = jnp.zeros((16,), jnp.int32)
        dloc_v[sl] = jnp.full((16,), SNR, jnp.int32)
        return _

    lax.fori_loop(0, SCAP // 16, init_list, 0)

    iot = lax.iota(jnp.int32, 16)

    def scan_chunk(c, cnt):
        pltpu.sync_copy(dst_hbm.at[pl.ds(c * ECH, ECH)], dst_v)

        def scan_grp(j, cnt):
            sl = pl.ds(j * 16, 16)
            dv = dst_v[sl]
            msk = (dv >= lo) & (dv < hi)
            ids = c * ECH + j * 16 + iot
            inc = msk.astype(jnp.int32)
            csum = plsc.cumsum(inc)
            pos = jnp.minimum(cnt, SCAP - 16) + csum - inc
            plsc.store_scatter(elist_v, [pos], ids, mask=msk)
            plsc.store_scatter(dloc_v, [pos], dv - lo, mask=msk)
            return cnt + jnp.sum(inc)

        return lax.fori_loop(0, ECH // 16, scan_grp, cnt)

    cnt = lax.fori_loop(0, E // ECH, scan_chunk, jnp.int32(0))
    cnt = jnp.minimum(cnt, SCAP)
    nch = (cnt + MCH - 1) // MCH

    def proc_chunk(q, _):
        pltpu.async_copy(m_hbm.at[elist_v.at[pl.ds(q * MCH, MCH)]],
                         rows_v, sem).wait()

        def rmw(t16, _):
            dlv = dloc_v[pl.ds(q * MCH + t16 * 16, 16)]
            for t in range(16):
                dl = dlv[t]
                for jj in range(4):
                    sl = pl.ds(jj * 16, 16)
                    acc_v[dl, sl] = jnp.maximum(acc_v[dl, sl],
                                                rows_v[t16 * 16 + t, sl])
            return _

        lax.fori_loop(0, MCH // 16, rmw, 0)
        return _

    lax.fori_loop(0, nch, proc_chunk, 0)

    def clean(i, _):
        for jj in range(4):
            sl = pl.ds(jj * 16, 16)
            v = acc_v[i, sl]
            rows_v[i, sl] = jnp.where(v == NEG_INF, 0.0, v)
        return _

    lax.fori_loop(0, SNR, clean, 0)
    pltpu.sync_copy(rows_v.at[pl.ds(0, SNR)], out_hbm.at[pl.ds(lo, SNR)])


@jax.jit
def _sc_segmax(m, dst):
    mesh = plsc.VectorSubcoreMesh(core_axis_name="c", subcore_axis_name="s")
    return pl.kernel(
        _segmax_body,
        out_type=jax.ShapeDtypeStruct((HNP, ML), jnp.float32),
        mesh=mesh,
        scratch_types=[
            pltpu.VMEM((SNR + 8, ML), jnp.float32),
            pltpu.VMEM((ECH,), jnp.int32),
            pltpu.VMEM((SCAP,), jnp.int32),
            pltpu.VMEM((SCAP,), jnp.int32),
            pltpu.VMEM((MCH, ML), jnp.float32),
            pltpu.SemaphoreType.DMA,
        ],
        compiler_params=pltpu.CompilerParams(use_tc_tiling_on_sc=False, needs_layout_passes=False),
    )(m, dst)


_TAKE_DN = lax.GatherDimensionNumbers(
    offset_dims=(), collapsed_slice_dims=(0,), start_index_map=(0,))


def _vtake(x, i):
    return lax.gather(x, i[:, None], _TAKE_DN, (1,),
                      mode=lax.GatherScatterMode.PROMISE_IN_BOUNDS)


def _seg_scan_last(key, val, neutral, op):
    iot = lax.iota(jnp.int32, 16)
    for s in (1, 2, 4, 8):
        pidx = jnp.maximum(iot - s, 0)
        pk = _vtake(key, pidx)
        pv = _vtake(val, pidx)
        ok = (pk == key) & (iot >= s)
        val = op(val, jnp.where(ok, pv, neutral))
    nk = _vtake(key, jnp.minimum(iot + 1, 15))
    last = (key != nk) | (iot == 15)
    return val, last


def _combine_partials(acc_v, shared, red_v, cmb_v, part_hbm, cid, sid, op):
    pltpu.sync_copy(acc_v, shared.at[sid])
    plsc.subcore_barrier()
    base = sid * 640
    pltpu.sync_copy(shared.at[0, pl.ds(base, 640)], red_v)
    for t in range(1, 16):
        pltpu.sync_copy(shared.at[t, pl.ds(base, 640)], cmb_v)

        def red_body(i, _):
            sl = pl.ds(i * 16, 16)
            red_v[sl] = op(red_v[sl], cmb_v[sl])
            return _

        lax.fori_loop(0, 40, red_body, 0)
    pltpu.sync_copy(red_v, part_hbm.at[cid, pl.ds(base, 640)])


def _heads_step_body(h_hbm, kn_hbm, lh_hbm, src_hbm, dst_hbm, part_hbm,
                     h_v, kn_v, acc_v, src_v, dst_v, l_v, red_v, cmb_v,
                     shared):
    cid = lax.axis_index("c")
    sid = lax.axis_index("s")
    w = sid * NC + cid
    pltpu.sync_copy(h_hbm, h_v)
    pltpu.sync_copy(kn_hbm, kn_v)

    def init_body(i, _):
        acc_v[pl.ds(i * 16, 16)] = jnp.full((16,), NEG_INF, jnp.float32)
        return _

    lax.fori_loop(0, HNP // 16, init_body, 0)

    base = w * (E // NW)
    is_h1 = base < E2
    sign = jnp.where(is_h1, 1.0, -1.0).astype(jnp.float32)
    loff = base - jnp.where(is_h1, 0, E2)
    for c in range(E // NW // ECH):
        pltpu.sync_copy(src_hbm.at[pl.ds(base + c * ECH, ECH)], src_v)
        pltpu.sync_copy(dst_hbm.at[pl.ds(base + c * ECH, ECH)], dst_v)
        pltpu.sync_copy(lh_hbm.at[pl.ds(loff + c * ECH, ECH)], l_v)

        def grp_body(j, _):
            sl = pl.ds(j * 16, 16)
            sv = src_v[sl]
            dv = dst_v[sl]
            lv = l_v[sl] * sign
            hs = plsc.load_gather(h_v, [sv])
            ks = plsc.load_gather(kn_v, [sv])
            cand = jnp.where(ks > 0, hs - lv, NEG_INF)
            key, val = plsc.sort_key_val(dv, cand)
            val, last = _seg_scan_last(key, val, NEG_INF, jnp.maximum)
            cur = plsc.load_gather(acc_v, [key])
            plsc.store_scatter(acc_v, [key], jnp.maximum(cur, val), mask=last)
            return _

        lax.fori_loop(0, ECH // 16, grp_body, 0)
    _combine_partials(acc_v, shared, red_v, cmb_v, part_hbm, cid, sid,
                      jnp.maximum)


@jax.jit
def _sc_heads(h, kn, lh, src, dst):
    mesh = plsc.VectorSubcoreMesh(core_axis_name="c", subcore_axis_name="s")
    return pl.kernel(
        _heads_step_body,
        out_type=jax.ShapeDtypeStruct((2, HNP), jnp.float32),
        mesh=mesh,
        scratch_types=[
            pltpu.VMEM((HNP,), jnp.float32),
            pltpu.VMEM((HNP,), jnp.float32),
            pltpu.VMEM((HNP,), jnp.float32),
            pltpu.VMEM((ECH,), jnp.int32),
            pltpu.VMEM((ECH,), jnp.int32),
            pltpu.VMEM((ECH,), jnp.float32),
            pltpu.VMEM((640,), jnp.float32),
            pltpu.VMEM((640,), jnp.float32),
            pltpu.VMEM_SHARED((16, HNP), jnp.float32),
        ],
        compiler_params=pltpu.CompilerParams(use_tc_tiling_on_sc=False, needs_layout_passes=False),
    )(h, kn, lh, src, dst)


def _dhat_body(qd_hbm, dst_hbm, part_hbm,
               acc_v, dst_v, q_v, red_v, cmb_v, shared):
    cid = lax.axis_index("c")
    sid = lax.axis_index("s")
    w = sid * NC + cid

    def init_body(i, _):
        acc_v[pl.ds(i * 16, 16)] = jnp.zeros((16,), jnp.float32)
        return _

    lax.fori_loop(0, HNP // 16, init_body, 0)

    base = w * (E // NW)
    is_h1 = base < E2
    sign = jnp.where(is_h1, 1.0, -1.0).astype(jnp.float32)
    qoff = base - jnp.where(is_h1, 0, E2)
    for c in range(E // NW // ECH):
        pltpu.sync_copy(dst_hbm.at[pl.ds(base + c * ECH, ECH)], dst_v)
        pltpu.sync_copy(qd_hbm.at[pl.ds(qoff + c * ECH, ECH)], q_v)

        def grp_body(j, _):
            sl = pl.ds(j * 16, 16)
            dv = dst_v[sl]
            qv = q_v[sl] * sign
            key, val = plsc.sort_key_val(dv, qv)
            val, last = _seg_scan_last(key, val, 0.0, jnp.add)
            cur = plsc.load_gather(acc_v, [key])
            plsc.store_scatter(acc_v, [key], cur + val, mask=last)
            return _

        lax.fori_loop(0, ECH // 16, grp_body, 0)
    _combine_partials(acc_v, shared, red_v, cmb_v, part_hbm, cid, sid,
                      jnp.add)


@jax.jit
def _sc_dhat(qd, dst):
    mesh = plsc.VectorSubcoreMesh(core_axis_name="c", subcore_axis_name="s")
    return pl.kernel(
        _dhat_body,
        out_type=jax.ShapeDtypeStruct((2, HNP), jnp.float32),
        mesh=mesh,
        scratch_types=[
            pltpu.VMEM((HNP,), jnp.float32),
            pltpu.VMEM((ECH,), jnp.int32),
            pltpu.VMEM((ECH,), jnp.float32),
            pltpu.VMEM((640,), jnp.float32),
            pltpu.VMEM((640,), jnp.float32),
            pltpu.VMEM_SHARED((16, HNP), jnp.float32),
        ],
        compiler_params=pltpu.CompilerParams(use_tc_tiling_on_sc=False, needs_layout_passes=False),
    )(qd, dst)


def _netflow_body(h_hbm, src_hbm, dst_hbm, dh_hbm, h_v, src_v, dst_v, dh_v):
    cid = lax.axis_index("c")
    sid = lax.axis_index("s")
    w = sid * NC + cid

    @pl.when(w < 16)
    def _():
        pltpu.sync_copy(h_hbm, h_v)
        base = w * (E2 // 16)
        for c in range(E2 // 16 // ECH):
            pltpu.sync_copy(src_hbm.at[pl.ds(base + c * ECH, ECH)], src_v)
            pltpu.sync_copy(dst_hbm.at[pl.ds(base + c * ECH, ECH)], dst_v)

            def grp_body(j, _):
                sl = pl.ds(j * 16, 16)
                hs = plsc.load_gather(h_v, [src_v[sl]])
                hd = plsc.load_gather(h_v, [dst_v[sl]])
                dh_v[sl] = hs - hd
                return _

            lax.fori_loop(0, ECH // 16, grp_body, 0)
            pltpu.sync_copy(dh_v, dh_hbm.at[pl.ds(base + c * ECH, ECH)])


@jax.jit
def _sc_netflow(h, src, dst):
    mesh = plsc.VectorSubcoreMesh(core_axis_name="c", subcore_axis_name="s")
    return pl.kernel(
        _netflow_body,
        out_type=jax.ShapeDtypeStruct((E2,), jnp.float32),
        mesh=mesh,
        scratch_types=[
            pltpu.VMEM((HNP,), jnp.float32),
            pltpu.VMEM((ECH,), jnp.int32),
            pltpu.VMEM((ECH,), jnp.int32),
            pltpu.VMEM((ECH,), jnp.float32),
        ],
        compiler_params=pltpu.CompilerParams(use_tc_tiling_on_sc=False, needs_layout_passes=False),
    )(h, src, dst)


def _edge_mlp_body(gs_ref, gd_ref, z_ref, a_ref, b_ref, c_ref, w2_ref, m_ref):
    pre = (gs_ref[...] @ a_ref[...] + gd_ref[...] @ b_ref[...]
           + z_ref[...] @ c_ref[...])
    m_ref[...] = jnp.maximum(pre, 0.0) @ w2_ref[...]


def _edge_mlp(gs, z, ew1, ew2):
    a, b, c = ew1[:ML], ew1[ML:2 * ML], ew1[2 * ML:]
    grid = E // EBLK
    g2 = grid // 2
    return pl.pallas_call(
        _edge_mlp_body,
        grid=(grid,),
        in_specs=[
            pl.BlockSpec((EBLK, ML), lambda i: (i, 0)),
            pl.BlockSpec((EBLK, ML), lambda i: ((i + g2) % grid, 0)),
            pl.BlockSpec((EBLK, ML), lambda i: (i, 0)),
            pl.BlockSpec((ML, ML), lambda i: (0, 0)),
            pl.BlockSpec((ML, ML), lambda i: (0, 0)),
            pl.BlockSpec((ML, ML), lambda i: (0, 0)),
            pl.BlockSpec((ML, ML), lambda i: (0, 0)),
        ],
        out_specs=pl.BlockSpec((EBLK, ML), lambda i: (i, 0)),
        out_shape=jax.ShapeDtypeStruct((E, ML), jnp.float32),
    )(gs, gs, z, a, b, c, ew2)


def _edge_mlp_noz_body(gs_ref, gd_ref, a_ref, b_ref, w2_ref, m_ref):
    pre = gs_ref[...] @ a_ref[...] + gd_ref[...] @ b_ref[...]
    m_ref[...] = jnp.maximum(pre, 0.0) @ w2_ref[...]


def _edge_mlp_noz(gs, ew1, ew2):
    a, b = ew1[:ML], ew1[ML:2 * ML]
    grid = E // EBLK
    g2 = grid // 2
    return pl.pallas_call(
        _edge_mlp_noz_body,
        grid=(grid,),
        in_specs=[
            pl.BlockSpec((EBLK, ML), lambda i: (i, 0)),
            pl.BlockSpec((EBLK, ML), lambda i: ((i + g2) % grid, 0)),
            pl.BlockSpec((ML, ML), lambda i: (0, 0)),
            pl.BlockSpec((ML, ML), lambda i: (0, 0)),
            pl.BlockSpec((ML, ML), lambda i: (0, 0)),
        ],
        out_specs=pl.BlockSpec((EBLK, ML), lambda i: (i, 0)),
        out_shape=jax.ShapeDtypeStruct((E, ML), jnp.float32),
    )(gs, gs, a, b, ew2)


def _node_mlp_body(g_ref, agg_ref, n1_ref, n2_ref, w2_ref, out_ref):
    pre = g_ref[...] @ n1_ref[...] + agg_ref[...] @ n2_ref[...]
    out_ref[...] = jnp.maximum(pre, 0.0) @ w2_ref[...]


def _node_mlp(g, agg, nw1, nw2):
    n1, n2 = nw1[:ML], nw1[ML:]
    grid = N // NBLK
    return pl.pallas_call(
        _node_mlp_body,
        grid=(grid,),
        in_specs=[
            pl.BlockSpec((NBLK, ML), lambda i: (i, 0)),
            pl.BlockSpec((NBLK, ML), lambda i: (i, 0)),
            pl.BlockSpec((ML, ML), lambda i: (0, 0)),
            pl.BlockSpec((ML, ML), lambda i: (0, 0)),
            pl.BlockSpec((ML, ML), lambda i: (0, 0)),
        ],
        out_specs=pl.BlockSpec((NBLK, ML), lambda i: (i, 0)),
        out_shape=jax.ShapeDtypeStruct((N, ML), jnp.float32),
    )(g, agg, n1, n2, nw2)


def _zbar_body(gs_ref, gd_ref, z_ref, z1_ref, z2_ref, z3_ref, f1_ref, f2_ref,
               c1_ref, c2_ref):
    zbar = (gs_ref[...] @ z1_ref[...] + gd_ref[...] @ z2_ref[...]
            + z_ref[...] @ z3_ref[...])
    c1_ref[...] = zbar @ f1_ref[...]
    c2_ref[...] = zbar @ f2_ref[...]


def _zbar_flows(gs, z, z_latent_W, flows_latent_W):
    z1, z2, z3 = z_latent_W[:ML], z_latent_W[ML:2 * ML], z_latent_W[2 * ML:]
    f1, f2 = flows_latent_W[:ML], flows_latent_W[ML:]
    grid = E // EBLK
    g2 = grid // 2
    c1, c2 = pl.pallas_call(
        _zbar_body,
        grid=(grid,),
        in_specs=[
            pl.BlockSpec((EBLK, ML), lambda i: (i, 0)),
            pl.BlockSpec((EBLK, ML), lambda i: ((i + g2) % grid, 0)),
            pl.BlockSpec((EBLK, ML), lambda i: (i, 0)),
            pl.BlockSpec((ML, ML), lambda i: (0, 0)),
            pl.BlockSpec((ML, ML), lambda i: (0, 0)),
            pl.BlockSpec((ML, ML), lambda i: (0, 0)),
            pl.BlockSpec((ML, 1), lambda i: (0, 0)),
            pl.BlockSpec((ML, 1), lambda i: (0, 0)),
        ],
        out_specs=[
            pl.BlockSpec((EBLK, 1), lambda i: (i, 0)),
            pl.BlockSpec((EBLK, 1), lambda i: (i, 0)),
        ],
        out_shape=[
            jax.ShapeDtypeStruct((E, 1), jnp.float32),
            jax.ShapeDtypeStruct((E, 1), jnp.float32),
        ],
    )(gs, gs, z, z1, z2, z3, f1, f2)
    return c1, c2


def _g_encode_body(p0_ref, p1_ref, ds_ref, res_ref, w_ref, g_ref):
    dhat = p0_ref[...] + p1_ref[...]
    g_ref[...] = (dhat @ w_ref[0:1, :] + ds_ref[...] @ w_ref[1:2, :]
                  + res_ref[...] @ w_ref[2:3, :])


def _g_encode(p0, p1, d_star, res, node_in_W):
    grid = N // NBLK
    return pl.pallas_call(
        _g_encode_body,
        grid=(grid,),
        in_specs=[
            pl.BlockSpec((NBLK, 1), lambda i: (i, 0)),
            pl.BlockSpec((NBLK, 1), lambda i: (i, 0)),
            pl.BlockSpec((NBLK, 1), lambda i: (i, 0)),
            pl.BlockSpec((NBLK, 1), lambda i: (i, 0)),
            pl.BlockSpec((3, ML), lambda i: (0, 0)),
        ],
        out_specs=pl.BlockSpec((NBLK, ML), lambda i: (i, 0)),
        out_shape=jax.ShapeDtypeStruct((N, ML), jnp.float32),
    )(p0, p1, d_star, res, node_in_W)


E2R = E2 // 128


def _qd_l_body(qd_ref, c1_ref, c2_ref, r_ref, qd_out, lh_out):
    q = qd_ref[...] + c1_ref[...] + c2_ref[...]
    qd_out[...] = q
    lh_out[...] = r_ref[...] * q * jnp.power(jnp.abs(q) + ZETA, 0.852)


def _qd_l(qd, c1, c2, r_half_w):
    c1w = c1[:E2].reshape(E2R, 128)
    c2w = c2[E2:].reshape(E2R, 128)
    return pl.pallas_call(
        _qd_l_body,
        grid=(1,),
        in_specs=[pl.BlockSpec((E2R, 128), lambda i: (0, 0))] * 4,
        out_specs=[pl.BlockSpec((E2R, 128), lambda i: (0, 0))] * 2,
        out_shape=[
            jax.ShapeDtypeStruct((E2R, 128), jnp.float32),
            jax.ShapeDtypeStruct((E2R, 128), jnp.float32),
        ],
    )(qd, c1w, c2w, r_half_w)


def _edge_mlp_qz_body(gs_ref, gd_ref, dh_ref, qd_ref, r_ref, a_ref, b_ref,
                      wz_ref, w2_ref, m_ref, *, g2):
    sgn = jnp.where(pl.program_id(0) < g2, 1.0, -1.0).astype(jnp.float32)
    dh = dh_ref[...]
    qt = jnp.sign(dh) * jnp.power(jnp.abs(dh) / r_ref[...] + ZETA, 1.0 / 1.852)
    z = sgn * (qt @ wz_ref[0:1, :] + qd_ref[...] @ wz_ref[1:2, :])
    pre = gs_ref[...] @ a_ref[...] + gd_ref[...] @ b_ref[...] + z
    m_ref[...] = jnp.maximum(pre, 0.0) @ w2_ref[...]


def _edge_mlp_qz(gs, dh, qd, r_half, ew1, ew2, wz):
    a, b = ew1[:ML], ew1[ML:2 * ML]
    grid = E // EBLK
    g2 = grid // 2
    return pl.pallas_call(
        functools.partial(_edge_mlp_qz_body, g2=g2),
        grid=(grid,),
        in_specs=[
            pl.BlockSpec((EBLK, ML), lambda i: (i, 0)),
            pl.BlockSpec((EBLK, ML), lambda i: ((i + g2) % grid, 0)),
            pl.BlockSpec((EBLK, 1), lambda i: (i % g2, 0)),
            pl.BlockSpec((EBLK, 1), lambda i: (i % g2, 0)),
            pl.BlockSpec((EBLK, 1), lambda i: (i % g2, 0)),
            pl.BlockSpec((ML, ML), lambda i: (0, 0)),
            pl.BlockSpec((ML, ML), lambda i: (0, 0)),
            pl.BlockSpec((2, ML), lambda i: (0, 0)),
            pl.BlockSpec((ML, ML), lambda i: (0, 0)),
        ],
        out_specs=pl.BlockSpec((EBLK, ML), lambda i: (i, 0)),
        out_shape=jax.ShapeDtypeStruct((E, ML), jnp.float32),
    )(gs, gs, dh, qd, r_half, a, b, wz, ew2)


def _head_update_body(h_ref, kn_ref, p0_ref, p1_ref, h_out, kn_out):
    seg = jnp.maximum(p0_ref[...], p1_ref[...])
    has = seg != NEG_INF
    h_out[...] = jnp.where(kn_ref[...] > 0, h_ref[...],
                           jnp.where(has, seg, 0.0))
    kn_out[...] = jnp.maximum(kn_ref[...], has.astype(jnp.float32))


NR = HNP // 128


def _head_update(h, kn, part):
    pw = part.reshape(2, NR, 128)
    return pl.pallas_call(
        _head_update_body,
        grid=(1,),
        in_specs=[pl.BlockSpec((NR, 128), lambda i: (0, 0))] * 4,
        out_specs=[pl.BlockSpec((NR, 128), lambda i: (0, 0))] * 2,
        out_shape=[
            jax.ShapeDtypeStruct((NR, 128), jnp.float32),
            jax.ShapeDtypeStruct((NR, 128), jnp.float32),
        ],
    )(h, kn, pw[0], pw[1])


def kernel(x, edge_index, edge_attr, node_in_W, edge_W, z_latent_W,
           flows_latent_W, gnn_params):
    src = edge_index[0]
    dst = edge_index[1]
    r_half = edge_attr[:E2, 0:1]
    r_half_w = r_half.reshape(E2R, 128)
    h_star = x[:, 0:1]
    d_star = x[:, 1:2]
    res_mask = x[:, 4:5]
    h_star_w = jnp.pad(x[:, 0], (0, HNP - N)).reshape(NR, 128)
    res_w = jnp.pad(x[:, 4], (0, HNP - N)).reshape(NR, 128)
    zeros_n1 = jnp.zeros((N, 1), jnp.float32)
    wz = edge_W @ gnn_params[0][0][2 * ML:]

    qd = jnp.zeros((E2R, 128), jnp.float32)
    part_d = None
    dh = None
    h = None
    for k in range(2):
        if k == 0:
            g = _g_encode(zeros_n1, zeros_n1, d_star, res_mask, node_in_W)
        else:
            g = _g_encode(part_d[0, :N].reshape(N, 1),
                          part_d[1, :N].reshape(N, 1),
                          d_star, res_mask, node_in_W)
        z = None
        for li, (ew1, ew2, nw1, nw2) in enumerate(gnn_params):
            gs = _sc_gather(g, src)
            if li == 0:
                if k == 0:
                    m = _edge_mlp_noz(gs, ew1, ew2)
                else:
                    m = _edge_mlp_qz(gs, dh.reshape(E2, 1),
                                     qd.reshape(E2, 1), r_half,
                                     ew1, ew2, wz)
            else:
                m = _edge_mlp(gs, z, ew1, ew2)
            agg = jax.ops.segment_max(m, dst, num_segments=N)
            agg = jnp.where(jnp.isfinite(agg), agg, 0.0)
            g = _node_mlp(g, agg, nw1, nw2)
            z = m
        c1, c2 = _zbar_flows(_sc_gather(g, src), z, z_latent_W, flows_latent_W)
        qd, lh = _qd_l(qd, c1, c2, r_half_w)
        h, kn = h_star_w, res_w
        for _ in range(DIA):
            part = _sc_heads(h.reshape(HNP), kn.reshape(HNP),
                             lh.reshape(E2), src, dst)
            h, kn = _head_update(h, kn, part)
        if k == 0:
            part_d = _sc_dhat(qd.reshape(E2), dst)
            dh = _sc_netflow(h.reshape(HNP), src, dst)
    return h.reshape(HNP, 1)[:N]


# speedup vs baseline: 1.0510x; 1.0008x over previous
import functools
import jax
import jax.numpy as jnp
from jax import lax
from jax.experimental import pallas as pl
from jax.experimental.pallas import tpu as pltpu
from jax.experimental.pallas import tpu_sc as plsc

N = 10000
E2 = 160000
E = 2 * E2
ML = 64
DIA = 2
ZETA = 1e-6

EBLK = 8000
NBLK = 10000

NC, NS = 2, 16
NW = NC * NS
GCHUNK = 400
HNP = 10240
ECH = 2000
NEG_INF = float("-inf")


def _sc_gather_body(table_hbm, idx_hbm, out_hbm, idx_v, rows_v, sem):
    wid = lax.axis_index("s") * NC + lax.axis_index("c")
    per_w = E // NW
    base = wid * per_w
    nch = per_w // GCHUNK
    pltpu.sync_copy(idx_hbm.at[pl.ds(base, GCHUNK)], idx_v.at[0])
    descs = [pltpu.async_copy(table_hbm.at[idx_v.at[0]], rows_v.at[0], sem)]
    for c in range(nch):
        b = c & 1
        if c + 1 < nch:
            nb = 1 - b
            off = base + (c + 1) * GCHUNK
            pltpu.sync_copy(idx_hbm.at[pl.ds(off, GCHUNK)], idx_v.at[nb])
            descs.append(pltpu.async_copy(table_hbm.at[idx_v.at[nb]],
                                          rows_v.at[nb], sem))
        descs[c].wait()
        pltpu.sync_copy(rows_v.at[b], out_hbm.at[pl.ds(base + c * GCHUNK,
                                                       GCHUNK)])


@jax.jit
def _sc_gather(table, idx):
    mesh = plsc.VectorSubcoreMesh(core_axis_name="c", subcore_axis_name="s")
    return pl.kernel(
        _sc_gather_body,
        out_type=jax.ShapeDtypeStruct((E, ML), jnp.float32),
        mesh=mesh,
        scratch_types=[
            pltpu.VMEM((2, GCHUNK), jnp.int32),
            pltpu.VMEM((2, GCHUNK, ML), jnp.float32),
            pltpu.SemaphoreType.DMA,
        ],
        compiler_params=pltpu.CompilerParams(use_tc_tiling_on_sc=False, needs_layout_passes=False),
    )(table, idx)


_TAKE_DN = lax.GatherDimensionNumbers(
    offset_dims=(), collapsed_slice_dims=(0,), start_index_map=(0,))


def _vtake(x, i):
    return lax.gather(x, i[:, None], _TAKE_DN, (1,),
                      mode=lax.GatherScatterMode.PROMISE_IN_BOUNDS)


def _seg_scan_last(key, val, neutral, op):
    iot = lax.iota(jnp.int32, 16)
    for s in (1, 2, 4, 8):
        pidx = jnp.maximum(iot - s, 0)
        pk = _vtake(key, pidx)
        pv = _vtake(val, pidx)
        ok = (pk == key) & (iot >= s)
        val = op(val, jnp.where(ok, pv, neutral))
    nk = _vtake(key, jnp.minimum(iot + 1, 15))
    last = (key != nk) | (iot == 15)
    return val, last


def _combine_partials(acc_v, shared, red_v, cmb_v, part_hbm, cid, sid, op):
    pltpu.sync_copy(acc_v, shared.at[sid])
    plsc.subcore_barrier()
    base = sid * 640
    pltpu.sync_copy(shared.at[0, pl.ds(base, 640)], red_v)
    for t in range(1, 16):
        pltpu.sync_copy(shared.at[t, pl.ds(base, 640)], cmb_v)

        def red_body(i, _):
            sl = pl.ds(i * 16, 16)
            red_v[sl] = op(red_v[sl], cmb_v[sl])
            return _

        lax.fori_loop(0, 40, red_body, 0)
    pltpu.sync_copy(red_v, part_hbm.at[cid, pl.ds(base, 640)])


def _heads_step_body(h_hbm, kn_hbm, lh_hbm, src_hbm, dst_hbm, part_hbm,
                     h_v, kn_v, acc_v, src_v, dst_v, l_v, red_v, cmb_v,
                     shared):
    cid = lax.axis_index("c")
    sid = lax.axis_index("s")
    w = sid * NC + cid
    pltpu.sync_copy(h_hbm, h_v)
    pltpu.sync_copy(kn_hbm, kn_v)

    def init_body(i, _):
        acc_v[pl.ds(i * 16, 16)] = jnp.full((16,), NEG_INF, jnp.float32)
        return _

    lax.fori_loop(0, HNP // 16, init_body, 0)

    base = w * (E // NW)
    is_h1 = base < E2
    sign = jnp.where(is_h1, 1.0, -1.0).astype(jnp.float32)
    loff = base - jnp.where(is_h1, 0, E2)
    for c in range(E // NW // ECH):
        pltpu.sync_copy(src_hbm.at[pl.ds(base + c * ECH, ECH)], src_v)
        pltpu.sync_copy(dst_hbm.at[pl.ds(base + c * ECH, ECH)], dst_v)
        pltpu.sync_copy(lh_hbm.at[pl.ds(loff + c * ECH, ECH)], l_v)

        def grp_body(j, _):
            sl = pl.ds(j * 16, 16)
            sv = src_v[sl]
            dv = dst_v[sl]
            lv = l_v[sl] * sign
            hs = plsc.load_gather(h_v, [sv])
            ks = plsc.load_gather(kn_v, [sv])
            cand = jnp.where(ks > 0, hs - lv, NEG_INF)
            key, val = plsc.sort_key_val(dv, cand)
            val, last = _seg_scan_last(key, val, NEG_INF, jnp.maximum)
            cur = plsc.load_gather(acc_v, [key])
            plsc.store_scatter(acc_v, [key], jnp.maximum(cur, val), mask=last)
            return _

        lax.fori_loop(0, ECH // 16, grp_body, 0)
    _combine_partials(acc_v, shared, red_v, cmb_v, part_hbm, cid, sid,
                      jnp.maximum)


@jax.jit
def _sc_heads(h, kn, lh, src, dst):
    mesh = plsc.VectorSubcoreMesh(core_axis_name="c", subcore_axis_name="s")
    return pl.kernel(
        _heads_step_body,
        out_type=jax.ShapeDtypeStruct((2, HNP), jnp.float32),
        mesh=mesh,
        scratch_types=[
            pltpu.VMEM((HNP,), jnp.float32),
            pltpu.VMEM((HNP,), jnp.float32),
            pltpu.VMEM((HNP,), jnp.float32),
            pltpu.VMEM((ECH,), jnp.int32),
            pltpu.VMEM((ECH,), jnp.int32),
            pltpu.VMEM((ECH,), jnp.float32),
            pltpu.VMEM((640,), jnp.float32),
            pltpu.VMEM((640,), jnp.float32),
            pltpu.VMEM_SHARED((16, HNP), jnp.float32),
        ],
        compiler_params=pltpu.CompilerParams(use_tc_tiling_on_sc=False, needs_layout_passes=False),
    )(h, kn, lh, src, dst)


def _dhat_body(qd_hbm, dst_hbm, part_hbm,
               acc_v, dst_v, q_v, red_v, cmb_v, shared):
    cid = lax.axis_index("c")
    sid = lax.axis_index("s")
    w = sid * NC + cid

    def init_body(i, _):
        acc_v[pl.ds(i * 16, 16)] = jnp.zeros((16,), jnp.float32)
        return _

    lax.fori_loop(0, HNP // 16, init_body, 0)

    base = w * (E // NW)
    is_h1 = base < E2
    sign = jnp.where(is_h1, 1.0, -1.0).astype(jnp.float32)
    qoff = base - jnp.where(is_h1, 0, E2)
    for c in range(E // NW // ECH):
        pltpu.sync_copy(dst_hbm.at[pl.ds(base + c * ECH, ECH)], dst_v)
        pltpu.sync_copy(qd_hbm.at[pl.ds(qoff + c * ECH, ECH)], q_v)

        def grp_body(j, _):
            sl = pl.ds(j * 16, 16)
            dv = dst_v[sl]
            qv = q_v[sl] * sign
            key, val = plsc.sort_key_val(dv, qv)
            val, last = _seg_scan_last(key, val, 0.0, jnp.add)
            cur = plsc.load_gather(acc_v, [key])
            plsc.store_scatter(acc_v, [key], cur + val, mask=last)
            return _

        lax.fori_loop(0, ECH // 16, grp_body, 0)
    _combine_partials(acc_v, shared, red_v, cmb_v, part_hbm, cid, sid,
                      jnp.add)


@jax.jit
def _sc_dhat(qd, dst):
    mesh = plsc.VectorSubcoreMesh(core_axis_name="c", subcore_axis_name="s")
    return pl.kernel(
        _dhat_body,
        out_type=jax.ShapeDtypeStruct((2, HNP), jnp.float32),
        mesh=mesh,
        scratch_types=[
            pltpu.VMEM((HNP,), jnp.float32),
            pltpu.VMEM((ECH,), jnp.int32),
            pltpu.VMEM((ECH,), jnp.float32),
            pltpu.VMEM((640,), jnp.float32),
            pltpu.VMEM((640,), jnp.float32),
            pltpu.VMEM_SHARED((16, HNP), jnp.float32),
        ],
        compiler_params=pltpu.CompilerParams(use_tc_tiling_on_sc=False, needs_layout_passes=False),
    )(qd, dst)


def _netflow_body(h_hbm, src_hbm, dst_hbm, dh_hbm, h_v, src_v, dst_v, dh_v):
    cid = lax.axis_index("c")
    sid = lax.axis_index("s")
    w = sid * NC + cid

    @pl.when(w < 16)
    def _():
        pltpu.sync_copy(h_hbm, h_v)
        base = w * (E2 // 16)
        for c in range(E2 // 16 // ECH):
            pltpu.sync_copy(src_hbm.at[pl.ds(base + c * ECH, ECH)], src_v)
            pltpu.sync_copy(dst_hbm.at[pl.ds(base + c * ECH, ECH)], dst_v)

            def grp_body(j, _):
                sl = pl.ds(j * 16, 16)
                hs = plsc.load_gather(h_v, [src_v[sl]])
                hd = plsc.load_gather(h_v, [dst_v[sl]])
                dh_v[sl] = hs - hd
                return _

            lax.fori_loop(0, ECH // 16, grp_body, 0)
            pltpu.sync_copy(dh_v, dh_hbm.at[pl.ds(base + c * ECH, ECH)])


@jax.jit
def _sc_netflow(h, src, dst):
    mesh = plsc.VectorSubcoreMesh(core_axis_name="c", subcore_axis_name="s")
    return pl.kernel(
        _netflow_body,
        out_type=jax.ShapeDtypeStruct((E2,), jnp.float32),
        mesh=mesh,
        scratch_types=[
            pltpu.VMEM((HNP,), jnp.float32),
            pltpu.VMEM((ECH,), jnp.int32),
            pltpu.VMEM((ECH,), jnp.int32),
            pltpu.VMEM((ECH,), jnp.float32),
        ],
        compiler_params=pltpu.CompilerParams(use_tc_tiling_on_sc=False, needs_layout_passes=False),
    )(h, src, dst)


def _edge_mlp_body(gs_ref, gd_ref, z_ref, a_ref, b_ref, c_ref, w2_ref, m_ref):
    pre = (gs_ref[...] @ a_ref[...] + gd_ref[...] @ b_ref[...]
           + z_ref[...] @ c_ref[...])
    m_ref[...] = jnp.maximum(pre, 0.0) @ w2_ref[...]


def _edge_mlp(gs, z, ew1, ew2):
    a, b, c = ew1[:ML], ew1[ML:2 * ML], ew1[2 * ML:]
    grid = E // EBLK
    g2 = grid // 2
    return pl.pallas_call(
        _edge_mlp_body,
        grid=(grid,),
        in_specs=[
            pl.BlockSpec((EBLK, ML), lambda i: (i, 0)),
            pl.BlockSpec((EBLK, ML), lambda i: ((i + g2) % grid, 0)),
            pl.BlockSpec((EBLK, ML), lambda i: (i, 0)),
            pl.BlockSpec((ML, ML), lambda i: (0, 0)),
            pl.BlockSpec((ML, ML), lambda i: (0, 0)),
            pl.BlockSpec((ML, ML), lambda i: (0, 0)),
            pl.BlockSpec((ML, ML), lambda i: (0, 0)),
        ],
        out_specs=pl.BlockSpec((EBLK, ML), lambda i: (i, 0)),
        out_shape=jax.ShapeDtypeStruct((E, ML), jnp.float32),
    )(gs, gs, z, a, b, c, ew2)


def _edge_mlp_noz_body(gs_ref, gd_ref, a_ref, b_ref, w2_ref, m_ref):
    pre = gs_ref[...] @ a_ref[...] + gd_ref[...] @ b_ref[...]
    m_ref[...] = jnp.maximum(pre, 0.0) @ w2_ref[...]


def _edge_mlp_noz(gs, ew1, ew2):
    a, b = ew1[:ML], ew1[ML:2 * ML]
    grid = E // EBLK
    g2 = grid // 2
    return pl.pallas_call(
        _edge_mlp_noz_body,
        grid=(grid,),
        in_specs=[
            pl.BlockSpec((EBLK, ML), lambda i: (i, 0)),
            pl.BlockSpec((EBLK, ML), lambda i: ((i + g2) % grid, 0)),
            pl.BlockSpec((ML, ML), lambda i: (0, 0)),
            pl.BlockSpec((ML, ML), lambda i: (0, 0)),
            pl.BlockSpec((ML, ML), lambda i: (0, 0)),
        ],
        out_specs=pl.BlockSpec((EBLK, ML), lambda i: (i, 0)),
        out_shape=jax.ShapeDtypeStruct((E, ML), jnp.float32),
    )(gs, gs, a, b, ew2)


def _node_mlp_body(g_ref, agg_ref, n1_ref, n2_ref, w2_ref, out_ref):
    pre = g_ref[...] @ n1_ref[...] + agg_ref[...] @ n2_ref[...]
    out_ref[...] = jnp.maximum(pre, 0.0) @ w2_ref[...]


def _node_mlp(g, agg, nw1, nw2):
    n1, n2 = nw1[:ML], nw1[ML:]
    grid = N // NBLK
    return pl.pallas_call(
        _node_mlp_body,
        grid=(grid,),
        in_specs=[
            pl.BlockSpec((NBLK, ML), lambda i: (i, 0)),
            pl.BlockSpec((NBLK, ML), lambda i: (i, 0)),
            pl.BlockSpec((ML, ML), lambda i: (0, 0)),
            pl.BlockSpec((ML, ML), lambda i: (0, 0)),
            pl.BlockSpec((ML, ML), lambda i: (0, 0)),
        ],
        out_specs=pl.BlockSpec((NBLK, ML), lambda i: (i, 0)),
        out_shape=jax.ShapeDtypeStruct((N, ML), jnp.float32),
    )(g, agg, n1, n2, nw2)


def _zbar_body(gs_ref, gd_ref, z_ref, z1_ref, z2_ref, z3_ref, f1_ref, f2_ref,
               c1_ref, c2_ref):
    zbar = (gs_ref[...] @ z1_ref[...] + gd_ref[...] @ z2_ref[...]
            + z_ref[...] @ z3_ref[...])
    c1_ref[...] = zbar @ f1_ref[...]
    c2_ref[...] = zbar @ f2_ref[...]


def _zbar_flows(gs, z, z_latent_W, flows_latent_W):
    z1, z2, z3 = z_latent_W[:ML], z_latent_W[ML:2 * ML], z_latent_W[2 * ML:]
    f1, f2 = flows_latent_W[:ML], flows_latent_W[ML:]
    grid = E // EBLK
    g2 = grid // 2
    c1, c2 = pl.pallas_call(
        _zbar_body,
        grid=(grid,),
        in_specs=[
            pl.BlockSpec((EBLK, ML), lambda i: (i, 0)),
            pl.BlockSpec((EBLK, ML), lambda i: ((i + g2) % grid, 0)),
            pl.BlockSpec((EBLK, ML), lambda i: (i, 0)),
            pl.BlockSpec((ML, ML), lambda i: (0, 0)),
            pl.BlockSpec((ML, ML), lambda i: (0, 0)),
            pl.BlockSpec((ML, ML), lambda i: (0, 0)),
            pl.BlockSpec((ML, 1), lambda i: (0, 0)),
            pl.BlockSpec((ML, 1), lambda i: (0, 0)),
        ],
        out_specs=[
            pl.BlockSpec((EBLK, 1), lambda i: (i, 0)),
            pl.BlockSpec((EBLK, 1), lambda i: (i, 0)),
        ],
        out_shape=[
            jax.ShapeDtypeStruct((E, 1), jnp.float32),
            jax.ShapeDtypeStruct((E, 1), jnp.float32),
        ],
    )(gs, gs, z, z1, z2, z3, f1, f2)
    return c1, c2


def _g_encode_body(p0_ref, p1_ref, ds_ref, res_ref, w_ref, g_ref):
    dhat = p0_ref[...] + p1_ref[...]
    g_ref[...] = (dhat @ w_ref[0:1, :] + ds_ref[...] @ w_ref[1:2, :]
                  + res_ref[...] @ w_ref[2:3, :])


def _g_encode(p0, p1, d_star, res, node_in_W):
    grid = N // NBLK
    return pl.pallas_call(
        _g_encode_body,
        grid=(grid,),
        in_specs=[
            pl.BlockSpec((NBLK, 1), lambda i: (i, 0)),
            pl.BlockSpec((NBLK, 1), lambda i: (i, 0)),
            pl.BlockSpec((NBLK, 1), lambda i: (i, 0)),
            pl.BlockSpec((NBLK, 1), lambda i: (i, 0)),
            pl.BlockSpec((3, ML), lambda i: (0, 0)),
        ],
        out_specs=pl.BlockSpec((NBLK, ML), lambda i: (i, 0)),
        out_shape=jax.ShapeDtypeStruct((N, ML), jnp.float32),
    )(p0, p1, d_star, res, node_in_W)


E2R = E2 // 128


def _qd_l_body(qd_ref, c1_ref, c2_ref, r_ref, qd_out, lh_out):
    q = qd_ref[...] + c1_ref[...] + c2_ref[...]
    qd_out[...] = q
    lh_out[...] = r_ref[...] * q * jnp.power(jnp.abs(q) + ZETA, 0.852)


def _qd_l(qd, c1, c2, r_half_w):
    c1w = c1[:E2].reshape(E2R, 128)
    c2w = c2[E2:].reshape(E2R, 128)
    return pl.pallas_call(
        _qd_l_body,
        grid=(1,),
        in_specs=[pl.BlockSpec((E2R, 128), lambda i: (0, 0))] * 4,
        out_specs=[pl.BlockSpec((E2R, 128), lambda i: (0, 0))] * 2,
        out_shape=[
            jax.ShapeDtypeStruct((E2R, 128), jnp.float32),
            jax.ShapeDtypeStruct((E2R, 128), jnp.float32),
        ],
    )(qd, c1w, c2w, r_half_w)


def _edge_mlp_qz_body(gs_ref, gd_ref, dh_ref, qd_ref, r_ref, a_ref, b_ref,
                      wz_ref, w2_ref, m_ref, *, g2):
    sgn = jnp.where(pl.program_id(0) < g2, 1.0, -1.0).astype(jnp.float32)
    dh = dh_ref[...]
    qt = jnp.sign(dh) * jnp.power(jnp.abs(dh) / r_ref[...] + ZETA, 1.0 / 1.852)
    z = sgn * (qt @ wz_ref[0:1, :] + qd_ref[...] @ wz_ref[1:2, :])
    pre = gs_ref[...] @ a_ref[...] + gd_ref[...] @ b_ref[...] + z
    m_ref[...] = jnp.maximum(pre, 0.0) @ w2_ref[...]


def _edge_mlp_qz(gs, dh, qd, r_half, ew1, ew2, wz):
    a, b = ew1[:ML], ew1[ML:2 * ML]
    grid = E // EBLK
    g2 = grid // 2
    return pl.pallas_call(
        functools.partial(_edge_mlp_qz_body, g2=g2),
        grid=(grid,),
        in_specs=[
            pl.BlockSpec((EBLK, ML), lambda i: (i, 0)),
            pl.BlockSpec((EBLK, ML), lambda i: ((i + g2) % grid, 0)),
            pl.BlockSpec((EBLK, 1), lambda i: (i % g2, 0)),
            pl.BlockSpec((EBLK, 1), lambda i: (i % g2, 0)),
            pl.BlockSpec((EBLK, 1), lambda i: (i % g2, 0)),
            pl.BlockSpec((ML, ML), lambda i: (0, 0)),
            pl.BlockSpec((ML, ML), lambda i: (0, 0)),
            pl.BlockSpec((2, ML), lambda i: (0, 0)),
            pl.BlockSpec((ML, ML), lambda i: (0, 0)),
        ],
        out_specs=pl.BlockSpec((EBLK, ML), lambda i: (i, 0)),
        out_shape=jax.ShapeDtypeStruct((E, ML), jnp.float32),
    )(gs, gs, dh, qd, r_half, a, b, wz, ew2)


def _head_update_body(h_ref, kn_ref, p0_ref, p1_ref, h_out, kn_out):
    seg = jnp.maximum(p0_ref[...], p1_ref[...])
    has = seg != NEG_INF
    h_out[...] = jnp.where(kn_ref[...] > 0, h_ref[...],
                           jnp.where(has, seg, 0.0))
    kn_out[...] = jnp.maximum(kn_ref[...], has.astype(jnp.float32))


NR = HNP // 128


def _head_update(h, kn, part):
    pw = part.reshape(2, NR, 128)
    return pl.pallas_call(
        _head_update_body,
        grid=(1,),
        in_specs=[pl.BlockSpec((NR, 128), lambda i: (0, 0))] * 4,
        out_specs=[pl.BlockSpec((NR, 128), lambda i: (0, 0))] * 2,
        out_shape=[
            jax.ShapeDtypeStruct((NR, 128), jnp.float32),
            jax.ShapeDtypeStruct((NR, 128), jnp.float32),
        ],
    )(h, kn, pw[0], pw[1])


def kernel(x, edge_index, edge_attr, node_in_W, edge_W, z_latent_W,
           flows_latent_W, gnn_params):
    src = edge_index[0]
    dst = edge_index[1]
    r_half = edge_attr[:E2, 0:1]
    r_half_w = r_half.reshape(E2R, 128)
    d_star = x[:, 1:2]
    res_mask = x[:, 4:5]
    h_star_w = jnp.pad(x[:, 0], (0, HNP - N)).reshape(NR, 128)
    res_w = jnp.pad(x[:, 4], (0, HNP - N)).reshape(NR, 128)
    zeros_n1 = jnp.zeros((N, 1), jnp.float32)
    wz = edge_W @ gnn_params[0][0][2 * ML:]

    qd = jnp.zeros((E2R, 128), jnp.float32)
    part_d = None
    dh = None
    h = None
    for k in range(2):
        if k == 0:
            g = _g_encode(zeros_n1, zeros_n1, d_star, res_mask, node_in_W)
        else:
            g = _g_encode(part_d[0, :N].reshape(N, 1),
                          part_d[1, :N].reshape(N, 1),
                          d_star, res_mask, node_in_W)
        z = None
        for li, (ew1, ew2, nw1, nw2) in enumerate(gnn_params):
            gs = _sc_gather(g, src)
            if li == 0:
                if k == 0:
                    m = _edge_mlp_noz(gs, ew1, ew2)
                else:
                    m = _edge_mlp_qz(gs, dh.reshape(E2, 1),
                                     qd.reshape(E2, 1), r_half,
                                     ew1, ew2, wz)
            else:
                m = _edge_mlp(gs, z, ew1, ew2)
            agg = jax.ops.segment_max(m, dst, num_segments=N)
            agg = jnp.where(jnp.isfinite(agg), agg, 0.0)
            g = _node_mlp(g, agg, nw1, nw2)
            z = m
        c1, c2 = _zbar_flows(_sc_gather(g, src), z, z_latent_W, flows_latent_W)
        qd, lh = _qd_l(qd, c1, c2, r_half_w)
        h, kn = h_star_w, res_w
        for _ in range(DIA):
            part = _sc_heads(h.reshape(HNP), kn.reshape(HNP),
                             lh.reshape(E2), src, dst)
            h, kn = _head_update(h, kn, part)
        if k == 0:
            part_d = _sc_dhat(qd.reshape(E2), dst)
            dh = _sc_netflow(h.reshape(HNP), src, dst)
    return h.reshape(HNP, 1)[:N]
